# baseline (speedup 1.0000x reference)
import functools

import jax
import jax.numpy as jnp
from jax import lax
from jax.experimental import pallas as pl
from jax.experimental.pallas import tpu as pltpu
from jax.experimental.pallas import tpu_sc as plsc


_BM = 256


def _p12_body(a1_ref, a2_ref, ue_ref, ie_ref, ieb_ref, ueb_ref,
              s1i_ref, g1u_ref, s12u_ref, g1i_s):
    nb = pl.num_programs(0) // 2
    i = pl.program_id(0)
    bm = a1_ref.shape[0]

    @pl.when(i < nb)
    def _():
        blk = jnp.dot(a1_ref[...], ue_ref[...],
                      preferred_element_type=jnp.float32)
        g1i_s[pl.ds(i * bm, bm), :] = blk
        s1i_ref[...] = 0.25 * (ieb_ref[...] + blk)

    @pl.when(i >= nb)
    def _():
        a = a2_ref[...]
        g1u = jnp.dot(a, ie_ref[...], preferred_element_type=jnp.float32)
        g2u = jnp.dot(a, g1i_s[...], preferred_element_type=jnp.float32)
        g1u_ref[...] = g1u
        s12u_ref[...] = 0.25 * (ueb_ref[...] + g1u + g2u)


def _p12(iu, ui, ue, ie, bm=_BM):
    m, k = iu.shape
    n = ue.shape[1]
    nb = m // bm
    return pl.pallas_call(
        _p12_body,
        grid=(2 * nb,),
        in_specs=[
            pl.BlockSpec((bm, k), lambda i: (jnp.minimum(i, nb - 1), 0)),
            pl.BlockSpec((bm, k), lambda i: (jnp.maximum(i - nb, 0), 0)),
            pl.BlockSpec((k, n), lambda i: (0, 0)),
            pl.BlockSpec((k, n), lambda i: (0, 0)),
            pl.BlockSpec((bm, n), lambda i: (jnp.minimum(i, nb - 1), 0)),
            pl.BlockSpec((bm, n), lambda i: (jnp.maximum(i - nb, 0), 0)),
        ],
        out_specs=(
            pl.BlockSpec((bm, n), lambda i: (jnp.minimum(i, nb - 1), 0)),
            pl.BlockSpec((bm, n), lambda i: (jnp.maximum(i - nb, 0), 0)),
            pl.BlockSpec((bm, n), lambda i: (jnp.maximum(i - nb, 0), 0)),
        ),
        out_shape=(
            jax.ShapeDtypeStruct((m, n), jnp.float32),
            jax.ShapeDtypeStruct((m, n), jnp.float32),
            jax.ShapeDtypeStruct((m, n), jnp.float32),
        ),
        scratch_shapes=[pltpu.VMEM((k, n), jnp.float32)],
    )(iu, ui, ue, ie, ie, ue)


def _pgm_body(idx_ref, a_hbm, x_ref, o_ref, abuf, sems, cnt):
    nsteps = pl.num_programs(0)
    i = pl.program_id(0)
    bm = o_ref.shape[0]

    def issue(step, buf):
        base = step * bm

        def body(t, c):
            g = base + t
            r = idx_ref[g]
            rp = idx_ref[jnp.maximum(g - 1, 0)]
            new = jnp.logical_or(g == 0, r != rp)

            @pl.when(new)
            def _():
                pltpu.make_async_copy(
                    a_hbm.at[pl.ds(r, 1)],
                    abuf.at[buf].at[pl.ds(t, 1)],
                    sems.at[buf],
                ).start()

            return c + new.astype(jnp.int32)

        cnt[buf] = lax.fori_loop(0, bm, body, 0, unroll=8)

    @pl.when(i == 0)
    def _():
        issue(0, 0)

    @pl.when(i + 1 < nsteps)
    def _():
        issue(i + 1, (i + 1) % 2)

    def wbody(t, _):
        pltpu.make_async_copy(
            a_hbm.at[pl.ds(0, 1)],
            abuf.at[i % 2].at[pl.ds(0, 1)],
            sems.at[i % 2],
        ).wait()
        return _

    lax.fori_loop(0, cnt[i % 2], wbody, 0)
    o_ref[...] = jnp.dot(abuf[i % 2], x_ref[...],
                         preferred_element_type=jnp.float32)


def _pgm(a, x, idx_sorted, bm=256):
    m, k = a.shape
    n = x.shape[1]
    r = idx_sorted.shape[0]
    grid_spec = pltpu.PrefetchScalarGridSpec(
        num_scalar_prefetch=1,
        grid=(r // bm,),
        in_specs=[
            pl.BlockSpec(memory_space=pl.ANY),
            pl.BlockSpec((k, n), lambda i, idx_ref: (0, 0)),
        ],
        out_specs=pl.BlockSpec((bm, n), lambda i, idx_ref: (i, 0)),
        scratch_shapes=[
            pltpu.VMEM((2, bm, k), jnp.float32),
            pltpu.SemaphoreType.DMA((2,)),
            pltpu.SMEM((2,), jnp.int32),
        ],
    )
    return pl.pallas_call(
        _pgm_body,
        grid_spec=grid_spec,
        out_shape=jax.ShapeDtypeStruct((r, n), jnp.float32),
    )(idx_sorted, a, x)


def _sc_gather(s12u, s1i, g3u_rows, g2i_rows, g3i_rows, jspad, idx3d):
    _, f = s12u.shape
    nw, six, lw = idx3d.shape
    info = plsc.get_sparse_core_info()
    assert nw == info.num_cores * info.num_subcores and six == 6
    b = nw * lw
    fj = jspad.shape[1]
    mesh = plsc.VectorSubcoreMesh(core_axis_name="c", subcore_axis_name="s")

    @functools.partial(
        pl.kernel,
        out_type=(
            jax.ShapeDtypeStruct((8 * b, f), jnp.float32),
            jax.ShapeDtypeStruct((b, fj), jnp.float32),
        ),
        mesh=mesh,
        scratch_types=[
            pltpu.VMEM((6, lw), jnp.int32),
            pltpu.VMEM((4 * lw, f), jnp.float32),
            pltpu.VMEM((lw, fj), jnp.float32),
            pltpu.SemaphoreType.DMA,
        ],
    )
    def gather_k(s12u_ref, s1i_ref, g3u_ref, g2i_ref, g3i_ref, js_ref,
                 idx_ref, out_ref, outjs_ref, idx_v, rows_v, js_v, sem):
        wid = lax.axis_index("s") * info.num_cores + lax.axis_index("c")
        pltpu.sync_copy(idx_ref.at[wid], idx_v)
        plan = (
            (0, s12u_ref, 0),
            (1, g3u_ref, 1),
            (2, s1i_ref, 2),
            (3, g2i_ref, 3),
            (4, g3i_ref, 3),
            (5, s1i_ref, 4),
            (6, g2i_ref, 5),
            (7, g3i_ref, 5),
        )
        for half in range(2):
            cps = [
                pltpu.async_copy(
                    tab.at[idx_v.at[jrow]],
                    rows_v.at[pl.ds(q * lw, lw)],
                    sem,
                )
                for q, (sec, tab, jrow) in enumerate(plan[4 * half:4 * half + 4])
            ]
            for c in cps:
                c.wait()
            for q, (sec, tab, jrow) in enumerate(plan[4 * half:4 * half + 4]):
                pltpu.sync_copy(
                    rows_v.at[pl.ds(q * lw, lw)],
                    out_ref.at[pl.ds(sec * b + wid * lw, lw)],
                )
        pltpu.async_copy(js_ref.at[idx_v.at[0]], js_v, sem).wait()
        pltpu.sync_copy(js_v, outjs_ref.at[pl.ds(wid * lw, lw)])

    return gather_k(s12u, s1i, g3u_rows, g2i_rows, g3i_rows, jspad, idx3d)


def _loss_body(rows_ref, js_ref, pi_ref, pj_ref, loss_ref, loss2_ref):
    b = pi_ref.shape[0]

    def sec(k):
        return rows_ref[pl.ds(k * b, b), :]

    u = sec(0) + js_ref[:, 0:1] * sec(1)
    ie = sec(2) + 0.25 * (sec(3) + sec(4))
    je = sec(5) + 0.25 * (sec(6) + sec(7))
    pi = jnp.sum(u * ie, axis=1)
    pj = jnp.sum(u * je, axis=1)
    pi_ref[...] = pi
    pj_ref[...] = pj
    d = pi - pj
    loss2 = jnp.mean(jnp.log(1.0 + jnp.exp(-d)))
    l2 = 0.0001 * jnp.sum(u * u + ie * ie + je * je, axis=1)
    loss2_ref[...] = jnp.reshape(loss2, (1, 1))
    loss_ref[...] = jnp.reshape(loss2 + jnp.mean(l2), (1, 1))


def _loss(rows, js):
    b = rows.shape[0] // 8
    return pl.pallas_call(
        _loss_body,
        out_shape=(
            jax.ShapeDtypeStruct((b,), jnp.float32),
            jax.ShapeDtypeStruct((b,), jnp.float32),
            jax.ShapeDtypeStruct((1, 1), jnp.float32),
            jax.ShapeDtypeStruct((1, 1), jnp.float32),
        ),
    )(rows, js)


@jax.jit
def kernel(user, item_i, item_j, user_item_3, item_user_3, user_js,
           embed_user_weight, embed_item_weight,
           user_item_matrix, item_user_matrix):
    ue = embed_user_weight
    ie = embed_item_weight
    nw = 32
    b = user.shape[0]
    lw = b // nw

    user = user.astype(jnp.int32)
    item_i = item_i.astype(jnp.int32)
    item_j = item_j.astype(jnp.int32)

    su = jnp.sort(user)
    user_pos = jnp.searchsorted(su, user).astype(jnp.int32)
    it_all = jnp.concatenate([item_i, item_j])
    si = jnp.sort(it_all)
    pos_i = jnp.searchsorted(si, item_i).astype(jnp.int32)
    pos_j = jnp.searchsorted(si, item_j).astype(jnp.int32)

    s1i, g1u, s12u = _p12(item_user_matrix, user_item_matrix, ue, ie)

    g3u_rows = _pgm(user_item_3, ie, su)
    g2i_rows = _pgm(item_user_matrix, g1u, si)
    g3i_rows = _pgm(item_user_3, ue, si)

    jspad = jnp.broadcast_to(user_js, (user_js.shape[0], 128))
    idx = jnp.stack(
        [user.reshape(nw, lw), user_pos.reshape(nw, lw),
         item_i.reshape(nw, lw), pos_i.reshape(nw, lw),
         item_j.reshape(nw, lw), pos_j.reshape(nw, lw)], axis=1)
    rows, jsr = _sc_gather(s12u, s1i, g3u_rows, g2i_rows, g3i_rows,
                           jspad, idx)

    pi, pj, loss, loss2 = _loss(rows, jsr)
    return pi, pj, loss[0, 0], loss2[0, 0]


# speedup vs baseline: 2.2997x; 2.2997x over previous
import functools

import jax
import jax.numpy as jnp
from jax import lax
from jax.experimental import pallas as pl
from jax.experimental.pallas import tpu as pltpu
from jax.experimental.pallas import tpu_sc as plsc


_BM = 256


def _p12_body(a1_ref, a2_ref, ue_ref, ie_ref, ieb_ref, ueb_ref,
              s1i_ref, g1u_ref, s12u_ref, g1i_s):
    nb = pl.num_programs(0) // 2
    i = pl.program_id(0)
    bm = a1_ref.shape[0]

    @pl.when(i < nb)
    def _():
        blk = jnp.dot(a1_ref[...], ue_ref[...],
                      preferred_element_type=jnp.float32)
        g1i_s[pl.ds(i * bm, bm), :] = blk
        s1i_ref[...] = 0.25 * (ieb_ref[...] + blk)

    @pl.when(i >= nb)
    def _():
        a = a2_ref[...]
        g1u = jnp.dot(a, ie_ref[...], preferred_element_type=jnp.float32)
        g2u = jnp.dot(a, g1i_s[...], preferred_element_type=jnp.float32)
        g1u_ref[...] = g1u
        s12u_ref[...] = 0.25 * (ueb_ref[...] + g1u + g2u)


def _p12(iu, ui, ue, ie, bm=_BM):
    m, k = iu.shape
    n = ue.shape[1]
    nb = m // bm
    return pl.pallas_call(
        _p12_body,
        grid=(2 * nb,),
        in_specs=[
            pl.BlockSpec((bm, k), lambda i: (jnp.minimum(i, nb - 1), 0)),
            pl.BlockSpec((bm, k), lambda i: (jnp.maximum(i - nb, 0), 0)),
            pl.BlockSpec((k, n), lambda i: (0, 0)),
            pl.BlockSpec((k, n), lambda i: (0, 0)),
            pl.BlockSpec((bm, n), lambda i: (jnp.minimum(i, nb - 1), 0)),
            pl.BlockSpec((bm, n), lambda i: (jnp.maximum(i - nb, 0), 0)),
        ],
        out_specs=(
            pl.BlockSpec((bm, n), lambda i: (jnp.minimum(i, nb - 1), 0)),
            pl.BlockSpec((bm, n), lambda i: (jnp.maximum(i - nb, 0), 0)),
            pl.BlockSpec((bm, n), lambda i: (jnp.maximum(i - nb, 0), 0)),
        ),
        out_shape=(
            jax.ShapeDtypeStruct((m, n), jnp.float32),
            jax.ShapeDtypeStruct((m, n), jnp.float32),
            jax.ShapeDtypeStruct((m, n), jnp.float32),
        ),
        scratch_shapes=[pltpu.VMEM((k, n), jnp.float32)],
    )(iu, ui, ue, ie, ie, ue)


def _pgm_body(idx_ref, a_hbm, x_ref, o_ref, abuf, sems):
    nsteps = pl.num_programs(0)
    i = pl.program_id(0)
    bm = o_ref.shape[0]

    def issue(step, buf):
        base = step * bm

        def body(t, _):
            r = idx_ref[base + t]
            pltpu.make_async_copy(
                a_hbm.at[pl.ds(r, 1)],
                abuf.at[buf].at[pl.ds(t, 1)],
                sems.at[buf],
            ).start()
            return 0

        lax.fori_loop(0, bm, body, 0, unroll=8)

    @pl.when(i == 0)
    def _():
        issue(0, 0)

    @pl.when(i + 1 < nsteps)
    def _():
        issue(i + 1, (i + 1) % 2)

    def wbody(t, _):
        pltpu.make_async_copy(
            a_hbm.at[pl.ds(0, 1)],
            abuf.at[i % 2].at[pl.ds(0, 1)],
            sems.at[i % 2],
        ).wait()
        return 0

    lax.fori_loop(0, bm, wbody, 0, unroll=8)
    o_ref[...] = jnp.dot(abuf[i % 2], x_ref[...],
                         preferred_element_type=jnp.float32)


def _pgm(a, x, idx, bm=256):
    m, k = a.shape
    n = x.shape[1]
    r = idx.shape[0]
    grid_spec = pltpu.PrefetchScalarGridSpec(
        num_scalar_prefetch=1,
        grid=(r // bm,),
        in_specs=[
            pl.BlockSpec(memory_space=pl.ANY),
            pl.BlockSpec((k, n), lambda i, idx_ref: (0, 0)),
        ],
        out_specs=pl.BlockSpec((bm, n), lambda i, idx_ref: (i, 0)),
        scratch_shapes=[
            pltpu.VMEM((2, bm, k), jnp.float32),
            pltpu.SemaphoreType.DMA((2,)),
        ],
    )
    return pl.pallas_call(
        _pgm_body,
        grid_spec=grid_spec,
        out_shape=jax.ShapeDtypeStruct((r, n), jnp.float32),
    )(idx, a, x)


def _p45_body(a1_ref, a2_ref, g1u_ref, ue_ref, s1i_ref, o_ref, g2i_s):
    nb = pl.num_programs(0) // 2
    i = pl.program_id(0)
    bm = a1_ref.shape[0]

    @pl.when(i < nb)
    def _():
        g2i_s[pl.ds(i * bm, bm), :] = jnp.dot(
            a1_ref[...], g1u_ref[...], preferred_element_type=jnp.float32)

    @pl.when(i >= nb)
    def _():
        g3 = jnp.dot(a2_ref[...], ue_ref[...],
                     preferred_element_type=jnp.float32)
        o_ref[...] = (s1i_ref[...]
                      + 0.25 * (g2i_s[pl.ds((i - nb) * bm, bm), :] + g3))


def _p45(iu, iu3, g1u, ue, s1i, bm=_BM):
    m, k = iu.shape
    n = ue.shape[1]
    nb = m // bm
    return pl.pallas_call(
        _p45_body,
        grid=(2 * nb,),
        in_specs=[
            pl.BlockSpec((bm, k), lambda i: (jnp.minimum(i, nb - 1), 0)),
            pl.BlockSpec((bm, k), lambda i: (jnp.maximum(i - nb, 0), 0)),
            pl.BlockSpec((k, n), lambda i: (0, 0)),
            pl.BlockSpec((k, n), lambda i: (0, 0)),
            pl.BlockSpec((bm, n), lambda i: (jnp.maximum(i - nb, 0), 0)),
        ],
        out_specs=pl.BlockSpec((bm, n), lambda i: (jnp.maximum(i - nb, 0), 0)),
        out_shape=jax.ShapeDtypeStruct((m, n), jnp.float32),
        scratch_shapes=[pltpu.VMEM((m, n), jnp.float32)],
    )(iu, iu3, g1u, ue, s1i)


def _sc_gather(s12u, jspad, gcn_items, idx3d):
    _, f = s12u.shape
    nw, three, lw = idx3d.shape
    info = plsc.get_sparse_core_info()
    assert nw == info.num_cores * info.num_subcores and three == 3
    b = nw * lw
    mesh = plsc.VectorSubcoreMesh(core_axis_name="c", subcore_axis_name="s")

    @functools.partial(
        pl.kernel,
        out_type=jax.ShapeDtypeStruct((4 * b, f), jnp.float32),
        mesh=mesh,
        scratch_types=[
            pltpu.VMEM((3, lw), jnp.int32),
            pltpu.VMEM((4 * lw, f), jnp.float32),
            pltpu.SemaphoreType.DMA,
        ],
    )
    def gather_k(s12u_ref, js_ref, itab_ref, idx_ref, out_ref,
                 idx_v, rows_v, sem):
        wid = lax.axis_index("s") * info.num_cores + lax.axis_index("c")
        pltpu.sync_copy(idx_ref.at[wid], idx_v)
        plan = (
            (s12u_ref, 0),
            (js_ref, 0),
            (itab_ref, 1),
            (itab_ref, 2),
        )
        cps = [
            pltpu.async_copy(
                tab.at[idx_v.at[jrow]],
                rows_v.at[pl.ds(q * lw, lw)],
                sem,
            )
            for q, (tab, jrow) in enumerate(plan)
        ]
        for c in cps:
            c.wait()
        for q in range(4):
            pltpu.sync_copy(
                rows_v.at[pl.ds(q * lw, lw)],
                out_ref.at[pl.ds(q * b + wid * lw, lw)],
            )

    return gather_k(s12u, jspad, gcn_items, idx3d)


def _loss_body(rows_ref, g3u_ref, pi_ref, pj_ref, loss_ref, loss2_ref):
    b = pi_ref.shape[0]

    def sec(k):
        return rows_ref[pl.ds(k * b, b), :]

    u = sec(0) + sec(1)[:, 0:1] * g3u_ref[...]
    ie = sec(2)
    je = sec(3)
    pi = jnp.sum(u * ie, axis=1)
    pj = jnp.sum(u * je, axis=1)
    pi_ref[...] = pi
    pj_ref[...] = pj
    d = pi - pj
    loss2 = jnp.mean(jnp.log(1.0 + jnp.exp(-d)))
    l2 = 0.0001 * jnp.sum(u * u + ie * ie + je * je, axis=1)
    loss2_ref[...] = jnp.reshape(loss2, (1, 1))
    loss_ref[...] = jnp.reshape(loss2 + jnp.mean(l2), (1, 1))


def _loss(rows, g3u_rows):
    b = rows.shape[0] // 4
    return pl.pallas_call(
        _loss_body,
        out_shape=(
            jax.ShapeDtypeStruct((b,), jnp.float32),
            jax.ShapeDtypeStruct((b,), jnp.float32),
            jax.ShapeDtypeStruct((1, 1), jnp.float32),
            jax.ShapeDtypeStruct((1, 1), jnp.float32),
        ),
    )(rows, g3u_rows)


@jax.jit
def kernel(user, item_i, item_j, user_item_3, item_user_3, user_js,
           embed_user_weight, embed_item_weight,
           user_item_matrix, item_user_matrix):
    ue = embed_user_weight
    ie = embed_item_weight
    nw = 32
    b = user.shape[0]
    lw = b // nw

    user = user.astype(jnp.int32)
    item_i = item_i.astype(jnp.int32)
    item_j = item_j.astype(jnp.int32)

    s1i, g1u, s12u = _p12(item_user_matrix, user_item_matrix, ue, ie)

    g3u_rows = _pgm(user_item_3, ie, user)

    gcn_items = _p45(item_user_matrix, item_user_3, g1u, ue, s1i)

    jspad = jnp.broadcast_to(user_js, (user_js.shape[0], 128))
    idx = jnp.stack(
        [user.reshape(nw, lw), item_i.reshape(nw, lw),
         item_j.reshape(nw, lw)], axis=1)
    rows = _sc_gather(s12u, jspad, gcn_items, idx)

    pi, pj, loss, loss2 = _loss(rows, g3u_rows)
    return pi, pj, loss[0, 0], loss2[0, 0]


# speedup vs baseline: 2.3442x; 1.0193x over previous
import functools

import jax
import jax.numpy as jnp
from jax import lax
from jax.experimental import pallas as pl
from jax.experimental.pallas import tpu as pltpu
from jax.experimental.pallas import tpu_sc as plsc


_BM = 256


def _p12_body(a1_ref, a2_ref, ue_ref, ie_ref, ieb_ref, ueb_ref,
              s1i_ref, g1u_ref, s12u_ref, g1i_s):
    nb = pl.num_programs(0) // 2
    i = pl.program_id(0)
    bm = a1_ref.shape[0]

    @pl.when(i < nb)
    def _():
        blk = jnp.dot(a1_ref[...], ue_ref[...],
                      preferred_element_type=jnp.float32)
        g1i_s[pl.ds(i * bm, bm), :] = blk
        s1i_ref[...] = 0.25 * (ieb_ref[...] + blk)

    @pl.when(i >= nb)
    def _():
        a = a2_ref[...]
        g1u = jnp.dot(a, ie_ref[...], preferred_element_type=jnp.float32)
        g2u = jnp.dot(a, g1i_s[...], preferred_element_type=jnp.float32)
        g1u_ref[...] = g1u
        s12u_ref[...] = 0.25 * (ueb_ref[...] + g1u + g2u)


def _p12(iu, ui, ue, ie, bm=_BM):
    m, k = iu.shape
    n = ue.shape[1]
    nb = m // bm
    return pl.pallas_call(
        _p12_body,
        grid=(2 * nb,),
        in_specs=[
            pl.BlockSpec((bm, k), lambda i: (jnp.minimum(i, nb - 1), 0)),
            pl.BlockSpec((bm, k), lambda i: (jnp.maximum(i - nb, 0), 0)),
            pl.BlockSpec((k, n), lambda i: (0, 0)),
            pl.BlockSpec((k, n), lambda i: (0, 0)),
            pl.BlockSpec((bm, n), lambda i: (jnp.minimum(i, nb - 1), 0)),
            pl.BlockSpec((bm, n), lambda i: (jnp.maximum(i - nb, 0), 0)),
        ],
        out_specs=(
            pl.BlockSpec((bm, n), lambda i: (jnp.minimum(i, nb - 1), 0)),
            pl.BlockSpec((bm, n), lambda i: (jnp.maximum(i - nb, 0), 0)),
            pl.BlockSpec((bm, n), lambda i: (jnp.maximum(i - nb, 0), 0)),
        ),
        out_shape=(
            jax.ShapeDtypeStruct((m, n), jnp.float32),
            jax.ShapeDtypeStruct((m, n), jnp.float32),
            jax.ShapeDtypeStruct((m, n), jnp.float32),
        ),
        scratch_shapes=[pltpu.VMEM((k, n), jnp.float32)],
    )(iu, ui, ue, ie, ie, ue)


def _pgm_body(idx_ref, a_hbm, x_ref, o_ref, abuf, sems):
    nsteps = pl.num_programs(0)
    i = pl.program_id(0)
    bm = o_ref.shape[0]

    def issue(step, buf):
        base = step * bm

        def body(t, _):
            r = idx_ref[base + t]
            pltpu.make_async_copy(
                a_hbm.at[pl.ds(r, 1)],
                abuf.at[buf].at[pl.ds(t, 1)],
                sems.at[buf],
            ).start()
            return 0

        lax.fori_loop(0, bm, body, 0, unroll=16)

    @pl.when(i == 0)
    def _():
        issue(0, 0)

    @pl.when(i + 1 < nsteps)
    def _():
        issue(i + 1, (i + 1) % 2)

    pltpu.make_async_copy(
        a_hbm.at[pl.ds(0, bm)],
        abuf.at[i % 2],
        sems.at[i % 2],
    ).wait()
    o_ref[...] = jnp.dot(abuf[i % 2], x_ref[...],
                         preferred_element_type=jnp.float32)


def _pgm(a, x, idx, bm=256):
    m, k = a.shape
    n = x.shape[1]
    r = idx.shape[0]
    grid_spec = pltpu.PrefetchScalarGridSpec(
        num_scalar_prefetch=1,
        grid=(r // bm,),
        in_specs=[
            pl.BlockSpec(memory_space=pl.ANY),
            pl.BlockSpec((k, n), lambda i, idx_ref: (0, 0)),
        ],
        out_specs=pl.BlockSpec((bm, n), lambda i, idx_ref: (i, 0)),
        scratch_shapes=[
            pltpu.VMEM((2, bm, k), jnp.float32),
            pltpu.SemaphoreType.DMA((2,)),
        ],
    )
    return pl.pallas_call(
        _pgm_body,
        grid_spec=grid_spec,
        out_shape=jax.ShapeDtypeStruct((r, n), jnp.float32),
    )(idx, a, x)


def _p45_body(a1_ref, a2_ref, g1u_ref, ue_ref, s1i_ref, o_ref, g2i_s):
    nb = pl.num_programs(0) // 2
    i = pl.program_id(0)
    bm = a1_ref.shape[0]

    @pl.when(i < nb)
    def _():
        g2i_s[pl.ds(i * bm, bm), :] = jnp.dot(
            a1_ref[...], g1u_ref[...], preferred_element_type=jnp.float32)

    @pl.when(i >= nb)
    def _():
        g3 = jnp.dot(a2_ref[...], ue_ref[...],
                     preferred_element_type=jnp.float32)
        o_ref[...] = (s1i_ref[...]
                      + 0.25 * (g2i_s[pl.ds((i - nb) * bm, bm), :] + g3))


def _p45(iu, iu3, g1u, ue, s1i, bm=_BM):
    m, k = iu.shape
    n = ue.shape[1]
    nb = m // bm
    return pl.pallas_call(
        _p45_body,
        grid=(2 * nb,),
        in_specs=[
            pl.BlockSpec((bm, k), lambda i: (jnp.minimum(i, nb - 1), 0)),
            pl.BlockSpec((bm, k), lambda i: (jnp.maximum(i - nb, 0), 0)),
            pl.BlockSpec((k, n), lambda i: (0, 0)),
            pl.BlockSpec((k, n), lambda i: (0, 0)),
            pl.BlockSpec((bm, n), lambda i: (jnp.maximum(i - nb, 0), 0)),
        ],
        out_specs=pl.BlockSpec((bm, n), lambda i: (jnp.maximum(i - nb, 0), 0)),
        out_shape=jax.ShapeDtypeStruct((m, n), jnp.float32),
        scratch_shapes=[pltpu.VMEM((m, n), jnp.float32)],
    )(iu, iu3, g1u, ue, s1i)


def _sc_gather(s12u, jspad, gcn_items, idx3d):
    _, f = s12u.shape
    nw, three, lw = idx3d.shape
    info = plsc.get_sparse_core_info()
    assert nw == info.num_cores * info.num_subcores and three == 3
    b = nw * lw
    mesh = plsc.VectorSubcoreMesh(core_axis_name="c", subcore_axis_name="s")

    @functools.partial(
        pl.kernel,
        out_type=jax.ShapeDtypeStruct((4 * b, f), jnp.float32),
        mesh=mesh,
        scratch_types=[
            pltpu.VMEM((3, lw), jnp.int32),
            pltpu.VMEM((4 * lw, f), jnp.float32),
            pltpu.SemaphoreType.DMA,
        ],
    )
    def gather_k(s12u_ref, js_ref, itab_ref, idx_ref, out_ref,
                 idx_v, rows_v, sem):
        wid = lax.axis_index("s") * info.num_cores + lax.axis_index("c")
        pltpu.sync_copy(idx_ref.at[wid], idx_v)
        plan = (
            (s12u_ref, 0),
            (js_ref, 0),
            (itab_ref, 1),
            (itab_ref, 2),
        )
        cps = [
            pltpu.async_copy(
                tab.at[idx_v.at[jrow]],
                rows_v.at[pl.ds(q * lw, lw)],
                sem,
            )
            for q, (tab, jrow) in enumerate(plan)
        ]
        for c in cps:
            c.wait()
        for q in range(4):
            pltpu.sync_copy(
                rows_v.at[pl.ds(q * lw, lw)],
                out_ref.at[pl.ds(q * b + wid * lw, lw)],
            )

    return gather_k(s12u, jspad, gcn_items, idx3d)


def _loss_body(rows_ref, g3u_ref, pi_ref, pj_ref, loss_ref, loss2_ref):
    b = pi_ref.shape[0]

    def sec(k):
        return rows_ref[pl.ds(k * b, b), :]

    u = sec(0) + sec(1)[:, 0:1] * g3u_ref[...]
    ie = sec(2)
    je = sec(3)
    pi = jnp.sum(u * ie, axis=1)
    pj = jnp.sum(u * je, axis=1)
    pi_ref[...] = pi
    pj_ref[...] = pj
    d = pi - pj
    loss2 = jnp.mean(jnp.log(1.0 + jnp.exp(-d)))
    l2 = 0.0001 * jnp.sum(u * u + ie * ie + je * je, axis=1)
    loss2_ref[...] = jnp.reshape(loss2, (1, 1))
    loss_ref[...] = jnp.reshape(loss2 + jnp.mean(l2), (1, 1))


def _loss(rows, g3u_rows):
    b = rows.shape[0] // 4
    return pl.pallas_call(
        _loss_body,
        out_shape=(
            jax.ShapeDtypeStruct((b,), jnp.float32),
            jax.ShapeDtypeStruct((b,), jnp.float32),
            jax.ShapeDtypeStruct((1, 1), jnp.float32),
            jax.ShapeDtypeStruct((1, 1), jnp.float32),
        ),
    )(rows, g3u_rows)


@jax.jit
def kernel(user, item_i, item_j, user_item_3, item_user_3, user_js,
           embed_user_weight, embed_item_weight,
           user_item_matrix, item_user_matrix):
    ue = embed_user_weight
    ie = embed_item_weight
    nw = 32
    b = user.shape[0]
    lw = b // nw

    user = user.astype(jnp.int32)
    item_i = item_i.astype(jnp.int32)
    item_j = item_j.astype(jnp.int32)

    s1i, g1u, s12u = _p12(item_user_matrix, user_item_matrix, ue, ie)

    g3u_rows = _pgm(user_item_3, ie, user)

    gcn_items = _p45(item_user_matrix, item_user_3, g1u, ue, s1i)

    jspad = jnp.broadcast_to(user_js, (user_js.shape[0], 128))
    idx = jnp.stack(
        [user.reshape(nw, lw), item_i.reshape(nw, lw),
         item_j.reshape(nw, lw)], axis=1)
    rows = _sc_gather(s12u, jspad, gcn_items, idx)

    pi, pj, loss, loss2 = _loss(rows, g3u_rows)
    return pi, pj, loss[0, 0], loss2[0, 0]
